# baseline (speedup 1.0000x reference)
import functools

import jax
import jax.numpy as jnp
from jax import lax
from jax.experimental import pallas as pl
from jax.experimental.pallas import tpu as pltpu
from jax.experimental.pallas import tpu_sc as plsc

N, K, D = 16384, 64, 64
BLK = 512
GB = 512
NGB = N // GB
NW = 32
PW = N // NW


def _stage_a(xt_ref, x_ref, c_ref, lt_ref,
             pos_ref, counts_ref, sums_ref, off_ref, xp_ref,
             pred_s, posl_s, *, nblk):
    p = pl.program_id(0)
    i = pl.program_id(1)

    @pl.when((p == 0) & (i == 0))
    def _init():
        counts_ref[:, :] = jnp.zeros_like(counts_ref)
        sums_ref[:, :] = jnp.zeros_like(sums_ref)

    kio = lax.broadcasted_iota(jnp.int32, (K, BLK), 0)

    @pl.when(p == 0)
    def _phase0():
        xt = xt_ref[:, :]
        xb = x_ref[:, :]
        c = c_ref[:, :]

        cn = jnp.sum(c * c, axis=1, keepdims=True)
        d2 = cn - 2.0 * jnp.dot(c, xt, preferred_element_type=jnp.float32)

        dmin = jnp.min(d2, axis=0, keepdims=True)
        pred = jnp.min(jnp.where(d2 <= dmin, kio, K), axis=0, keepdims=True)
        oh = (kio == pred).astype(jnp.float32)
        oh_bf = oh.astype(jnp.bfloat16)

        cum = jnp.dot(oh_bf, lt_ref[:, :], preferred_element_type=jnp.float32)
        carry = counts_ref[:, :]
        chi = jnp.floor(carry * (1.0 / 128.0))
        clo = carry - chi * 128.0
        cumhi = jnp.floor(cum * (1.0 / 128.0))
        cumlo = cum - cumhi * 128.0
        hi_t = (cumhi + chi).astype(jnp.bfloat16)
        lo_t = (cumlo + clo).astype(jnp.bfloat16)
        ones_bf = jnp.ones((1, K), jnp.bfloat16)
        posl = (128.0 * jnp.dot(ones_bf, oh_bf * hi_t,
                                preferred_element_type=jnp.float32)
                + jnp.dot(ones_bf, oh_bf * lo_t,
                          preferred_element_type=jnp.float32))

        pred_s[pl.ds(i, 1), :] = pred
        posl_s[pl.ds(i, 1), :] = posl.astype(jnp.int32)

        counts_ref[:, :] = carry + jnp.dot(oh_bf,
                                           jnp.ones((BLK, 1), jnp.bfloat16),
                                           preferred_element_type=jnp.float32)
        sums_ref[:, :] += jnp.dot(oh_bf, xb.astype(jnp.bfloat16),
                                  preferred_element_type=jnp.float32)

        @pl.when(i == nblk - 1)
        def _epilogue():
            lk = (lax.broadcasted_iota(jnp.int32, (K, K), 1)
                  < lax.broadcasted_iota(jnp.int32, (K, K), 0)
                  ).astype(jnp.float32)
            off_ref[:, :] = jnp.dot(lk, counts_ref[:, :],
                                    preferred_element_type=jnp.float32)

    @pl.when(p == 1)
    def _phase1():
        predb = pred_s[pl.ds(i, 1), :]
        poslb = posl_s[pl.ds(i, 1), :]
        ohb1 = kio == predb
        oh_bf = ohb1.astype(jnp.bfloat16)
        off = off_ref[:, :]
        ohi = jnp.floor(off * (1.0 / 128.0))
        olo = off - ohi * 128.0
        ones_bf = jnp.ones((1, K), jnp.bfloat16)
        offb = (128.0 * jnp.dot(ones_bf, oh_bf * ohi.astype(jnp.bfloat16),
                                preferred_element_type=jnp.float32)
                + jnp.dot(ones_bf, oh_bf * olo.astype(jnp.bfloat16),
                          preferred_element_type=jnp.float32))
        pos = poslb + offb.astype(jnp.int32)
        pos_ref[:, :, :] = pos.reshape(1, 1, BLK)
        xb = x_ref[:, :]
        xp_ref[:, :] = jnp.concatenate([xb, jnp.zeros_like(xb)], axis=1)


def _run_stage_a(x, xt, centers, lt):
    nblk = N // BLK
    return pl.pallas_call(
        functools.partial(_stage_a, nblk=nblk),
        grid=(2, nblk),
        in_specs=[
            pl.BlockSpec((D, BLK), lambda p, i: (0, i)),
            pl.BlockSpec((BLK, D), lambda p, i: (i, 0)),
            pl.BlockSpec((K, D), lambda p, i: (0, 0)),
            pl.BlockSpec((BLK, BLK), lambda p, i: (0, 0)),
        ],
        out_specs=[
            pl.BlockSpec((1, 1, BLK),
                         lambda p, i: (jnp.where(p == 0, 0, i), 0, 0)),
            pl.BlockSpec((K, 1), lambda p, i: (0, 0)),
            pl.BlockSpec((K, D), lambda p, i: (0, 0)),
            pl.BlockSpec((K, 1), lambda p, i: (0, 0)),
            pl.BlockSpec((BLK, 2 * D),
                         lambda p, i: (jnp.where(p == 1, i, 0), 0)),
        ],
        out_shape=[
            jax.ShapeDtypeStruct((nblk, 1, BLK), jnp.int32),
            jax.ShapeDtypeStruct((K, 1), jnp.float32),
            jax.ShapeDtypeStruct((K, D), jnp.float32),
            jax.ShapeDtypeStruct((K, 1), jnp.float32),
            jax.ShapeDtypeStruct((N, 2 * D), jnp.float32),
        ],
        scratch_shapes=[
            pltpu.VMEM((nblk, BLK), jnp.int32),
            pltpu.VMEM((nblk, BLK), jnp.int32),
        ],
        compiler_params=pltpu.CompilerParams(
            dimension_semantics=("arbitrary", "arbitrary"),
        ),
    )(xt, x, centers, lt)


def _scatter_body(x_hbm, pos_hbm, out_hbm, idx_v, rows_v, sem):
    wid = lax.axis_index("s") * 2 + lax.axis_index("c")
    base = wid * PW
    pltpu.sync_copy(pos_hbm.at[pl.ds(base, PW)], idx_v)
    pltpu.sync_copy(x_hbm.at[pl.ds(base, PW)], rows_v)
    pltpu.async_copy(rows_v, out_hbm.at[idx_v], sem).wait()


def _scatter_rows(xp, pos_i):
    mesh = plsc.VectorSubcoreMesh(core_axis_name="c", subcore_axis_name="s")
    fn = pl.kernel(
        _scatter_body,
        out_type=jax.ShapeDtypeStruct((N, 2 * D), jnp.float32),
        mesh=mesh,
        scratch_types=[
            pltpu.VMEM((PW,), jnp.int32),
            pltpu.VMEM((PW, 2 * D), jnp.float32),
            pltpu.SemaphoreType.DMA,
        ],
    )
    return fn(xp, pos_i)


def _gram_body(xs_ref, p_ref, acc):
    b = pl.program_id(0)

    @pl.when(b == 0)
    def _init():
        acc[:, :] = jnp.zeros_like(acc)

    p_ref[:, :] = acc[:, :]
    xsb = xs_ref[:, :D].astype(jnp.bfloat16)
    acc[:, :] += lax.dot_general(xsb, xsb, (((0,), (0,)), ((), ())),
                                 preferred_element_type=jnp.float32)


def _run_gram(xs):
    return pl.pallas_call(
        _gram_body,
        grid=(NGB,),
        in_specs=[pl.BlockSpec((GB, 2 * D), lambda b: (b, 0))],
        out_specs=pl.BlockSpec((D, D), lambda b: (b, 0)),
        out_shape=jax.ShapeDtypeStruct((NGB * D, D), jnp.float32),
        scratch_shapes=[pltpu.VMEM((D, D), jnp.float32)],
        compiler_params=pltpu.CompilerParams(
            dimension_semantics=("arbitrary",),
        ),
    )(xs)


def _stage_d(bep_ref, mlim_ref, xs_ref, p_ref,
             counts_ref, sums_ref, ft_ref, mt_ref, ct_ref, out_ref,
             sedge, prevt):
    k = pl.program_id(0)

    @pl.when(k == 0)
    def _init():
        prevt[:, :] = jnp.zeros_like(prevt)

    xsb = xs_ref[:, :D]
    mlim = mlim_ref[k]
    msk = (lax.broadcasted_iota(jnp.int32, (GB, 1), 0) < mlim
           ).astype(jnp.float32)
    xs_bf = xsb.astype(jnp.bfloat16)
    xm_bf = (xsb * msk).astype(jnp.bfloat16)
    pe = lax.dot_general(xm_bf, xs_bf, (((0,), (0,)), ((), ())),
                         preferred_element_type=jnp.float32)
    tk = p_ref[:, :] + pe
    sedge[pl.ds(k * D, D), :] = tk - prevt[:, :]
    prevt[:, :] = tk

    @pl.when(k == K - 1)
    def _epilogue():
        s_flat = sedge[:, :]
        counts = counts_ref[:, :]
        safe = jnp.maximum(counts, 1.0)
        means = sums_ref[:, :] / safe

        filling = counts / jnp.float32(N)
        loss_fil = jnp.sum((filling - ft_ref[:, :]) ** 2,
                           axis=(0, 1), keepdims=True) / jnp.float32(K)
        loss_means = jnp.sum((means - mt_ref[:, :]) ** 2,
                             axis=(0, 1), keepdims=True) / jnp.float32(K * D)

        m3 = jnp.reshape(jnp.broadcast_to(means[:, None, :], (K, D, D)),
                         (K * D, D))
        rio = lax.broadcasted_iota(jnp.int32, (K * D, D), 0)
        jio = lax.broadcasted_iota(jnp.int32, (K * D, D), 1)
        isel = (rio % D == jio).astype(jnp.float32)
        m4 = jnp.sum(m3 * isel, axis=1, keepdims=True)

        countsb = jnp.reshape(jnp.broadcast_to(counts[:, :, None], (K, D, 1)),
                              (K * D, 1))
        denomb = jnp.maximum(countsb - 1.0, 1.0)
        covs = (s_flat - countsb * (m4 * m3)) / denomb
        loss_covs = jnp.sum((covs - ct_ref[:, :]) ** 2,
                            axis=(0, 1), keepdims=True) / jnp.float32(K * D * D)

        out_ref[:, :] = loss_fil + loss_means + loss_covs


def _run_stage_d(bep, mlim, xs, pgram, counts, sums, ft, mt, ct):
    grid_spec = pltpu.PrefetchScalarGridSpec(
        num_scalar_prefetch=2,
        grid=(K,),
        in_specs=[
            pl.BlockSpec((GB, 2 * D), lambda k, bep, mlim: (bep[k], 0)),
            pl.BlockSpec((D, D), lambda k, bep, mlim: (bep[k], 0)),
            pl.BlockSpec((K, 1), lambda k, bep, mlim: (0, 0)),
            pl.BlockSpec((K, D), lambda k, bep, mlim: (0, 0)),
            pl.BlockSpec((K, 1), lambda k, bep, mlim: (0, 0)),
            pl.BlockSpec((K, D), lambda k, bep, mlim: (0, 0)),
            pl.BlockSpec((K * D, D), lambda k, bep, mlim: (0, 0)),
        ],
        out_specs=pl.BlockSpec((1, 1), lambda k, bep, mlim: (0, 0)),
        scratch_shapes=[
            pltpu.VMEM((K * D, D), jnp.float32),
            pltpu.VMEM((D, D), jnp.float32),
        ],
    )
    return pl.pallas_call(
        _stage_d,
        grid_spec=grid_spec,
        out_shape=jax.ShapeDtypeStruct((1, 1), jnp.float32),
        compiler_params=pltpu.CompilerParams(
            dimension_semantics=("arbitrary",),
        ),
    )(bep, mlim, xs, pgram, counts, sums, ft, mt, ct)


def kernel(x, cluster_centers, filling_target, means_target, covs_target):
    xt = x.T
    jio = lax.broadcasted_iota(jnp.int32, (BLK, BLK), 1)
    iio = lax.broadcasted_iota(jnp.int32, (BLK, BLK), 0)
    lt = (iio < jio).astype(jnp.bfloat16)

    pos3, counts, sums, off, xp = _run_stage_a(x, xt, cluster_centers, lt)

    pos_i = pos3.reshape(N)
    off_i = off.astype(jnp.int32).reshape(K)
    xs = _scatter_rows(xp, pos_i)

    pgram = _run_gram(xs)

    e_i = off_i + counts.astype(jnp.int32).reshape(K)
    bep = jnp.clip(e_i // GB, 0, NGB - 1).astype(jnp.int32)
    mlim = (e_i - bep * GB).astype(jnp.int32)

    ft = filling_target.reshape(K, 1)
    ct = covs_target.reshape(K * D, D)
    out = _run_stage_d(bep, mlim, xs, pgram, counts, sums,
                       ft, means_target, ct)
    return out[0, 0]


# speedup vs baseline: 1.0117x; 1.0117x over previous
import functools

import jax
import jax.numpy as jnp
from jax import lax
from jax.experimental import pallas as pl
from jax.experimental.pallas import tpu as pltpu
from jax.experimental.pallas import tpu_sc as plsc

N, K, D = 16384, 64, 64
BLK = 512
GB = 512
NGB = N // GB
NW = 32
PW = N // NW


def _stage_a(x_ref, c_ref, lt_ref,
             pos_ref, counts_ref, sums_ref, off_ref, xp_ref,
             pred_s, posl_s, *, nblk):
    p = pl.program_id(0)
    i = pl.program_id(1)

    @pl.when((p == 0) & (i == 0))
    def _init():
        counts_ref[:, :] = jnp.zeros_like(counts_ref)
        sums_ref[:, :] = jnp.zeros_like(sums_ref)

    kio = lax.broadcasted_iota(jnp.int32, (K, BLK), 0)

    @pl.when(p == 0)
    def _phase0():
        xb = x_ref[:, :]
        c = c_ref[:, :]

        cn = jnp.sum(c * c, axis=1, keepdims=True)
        d2 = cn - 2.0 * lax.dot_general(
            c, xb, (((1,), (1,)), ((), ())),
            preferred_element_type=jnp.float32)

        dmin = jnp.min(d2, axis=0, keepdims=True)
        pred = jnp.min(jnp.where(d2 <= dmin, kio, K), axis=0, keepdims=True)
        oh = (kio == pred).astype(jnp.float32)
        oh_bf = oh.astype(jnp.bfloat16)

        cum = jnp.dot(oh_bf, lt_ref[:, :], preferred_element_type=jnp.float32)
        carry = counts_ref[:, :]
        chi = jnp.floor(carry * (1.0 / 128.0))
        clo = carry - chi * 128.0
        cumhi = jnp.floor(cum * (1.0 / 128.0))
        cumlo = cum - cumhi * 128.0
        hi_t = (cumhi + chi).astype(jnp.bfloat16)
        lo_t = (cumlo + clo).astype(jnp.bfloat16)
        ones_bf = jnp.ones((1, K), jnp.bfloat16)
        posl = (128.0 * jnp.dot(ones_bf, oh_bf * hi_t,
                                preferred_element_type=jnp.float32)
                + jnp.dot(ones_bf, oh_bf * lo_t,
                          preferred_element_type=jnp.float32))

        pred_s[pl.ds(i, 1), :] = pred
        posl_s[pl.ds(i, 1), :] = posl.astype(jnp.int32)

        counts_ref[:, :] = carry + jnp.dot(oh_bf,
                                           jnp.ones((BLK, 1), jnp.bfloat16),
                                           preferred_element_type=jnp.float32)
        sums_ref[:, :] += jnp.dot(oh_bf, xb.astype(jnp.bfloat16),
                                  preferred_element_type=jnp.float32)

        @pl.when(i == nblk - 1)
        def _epilogue():
            lk = (lax.broadcasted_iota(jnp.int32, (K, K), 1)
                  < lax.broadcasted_iota(jnp.int32, (K, K), 0)
                  ).astype(jnp.float32)
            off_ref[:, :] = jnp.dot(lk, counts_ref[:, :],
                                    preferred_element_type=jnp.float32)

    @pl.when(p == 1)
    def _phase1():
        predb = pred_s[pl.ds(i, 1), :]
        poslb = posl_s[pl.ds(i, 1), :]
        ohb1 = kio == predb
        oh_bf = ohb1.astype(jnp.bfloat16)
        off = off_ref[:, :]
        ohi = jnp.floor(off * (1.0 / 128.0))
        olo = off - ohi * 128.0
        ones_bf = jnp.ones((1, K), jnp.bfloat16)
        offb = (128.0 * jnp.dot(ones_bf, oh_bf * ohi.astype(jnp.bfloat16),
                                preferred_element_type=jnp.float32)
                + jnp.dot(ones_bf, oh_bf * olo.astype(jnp.bfloat16),
                          preferred_element_type=jnp.float32))
        pos = poslb + offb.astype(jnp.int32)
        pos_ref[:, :, :] = pos.reshape(1, 1, BLK)
        xb = x_ref[:, :]
        xp_ref[:, :] = jnp.concatenate([xb, jnp.zeros_like(xb)], axis=1)


def _run_stage_a(x, centers, lt):
    nblk = N // BLK
    return pl.pallas_call(
        functools.partial(_stage_a, nblk=nblk),
        grid=(2, nblk),
        in_specs=[
            pl.BlockSpec((BLK, D), lambda p, i: (i, 0)),
            pl.BlockSpec((K, D), lambda p, i: (0, 0)),
            pl.BlockSpec((BLK, BLK), lambda p, i: (0, 0)),
        ],
        out_specs=[
            pl.BlockSpec((1, 1, BLK),
                         lambda p, i: (jnp.where(p == 0, 0, i), 0, 0)),
            pl.BlockSpec((K, 1), lambda p, i: (0, 0)),
            pl.BlockSpec((K, D), lambda p, i: (0, 0)),
            pl.BlockSpec((K, 1), lambda p, i: (0, 0)),
            pl.BlockSpec((BLK, 2 * D),
                         lambda p, i: (jnp.where(p == 1, i, 0), 0)),
        ],
        out_shape=[
            jax.ShapeDtypeStruct((nblk, 1, BLK), jnp.int32),
            jax.ShapeDtypeStruct((K, 1), jnp.float32),
            jax.ShapeDtypeStruct((K, D), jnp.float32),
            jax.ShapeDtypeStruct((K, 1), jnp.float32),
            jax.ShapeDtypeStruct((N, 2 * D), jnp.float32),
        ],
        scratch_shapes=[
            pltpu.VMEM((nblk, BLK), jnp.int32),
            pltpu.VMEM((nblk, BLK), jnp.int32),
        ],
        compiler_params=pltpu.CompilerParams(
            dimension_semantics=("arbitrary", "arbitrary"),
        ),
    )(x, centers, lt)


def _scatter_body(x_hbm, pos_hbm, out_hbm, idx_v, rows_v, sem):
    wid = lax.axis_index("s") * 2 + lax.axis_index("c")
    base = wid * PW
    pltpu.sync_copy(pos_hbm.at[pl.ds(base, PW)], idx_v)
    pltpu.sync_copy(x_hbm.at[pl.ds(base, PW)], rows_v)
    pltpu.async_copy(rows_v, out_hbm.at[idx_v], sem).wait()


def _scatter_rows(xp, pos_i):
    mesh = plsc.VectorSubcoreMesh(core_axis_name="c", subcore_axis_name="s")
    fn = pl.kernel(
        _scatter_body,
        out_type=jax.ShapeDtypeStruct((N, 2 * D), jnp.float32),
        mesh=mesh,
        scratch_types=[
            pltpu.VMEM((PW,), jnp.int32),
            pltpu.VMEM((PW, 2 * D), jnp.float32),
            pltpu.SemaphoreType.DMA,
        ],
    )
    return fn(xp, pos_i)


def _gram_body(xs_ref, p_ref, acc):
    b = pl.program_id(0)

    @pl.when(b == 0)
    def _init():
        acc[:, :] = jnp.zeros_like(acc)

    p_ref[:, :] = acc[:, :]
    xsb = xs_ref[:, :D].astype(jnp.bfloat16)
    acc[:, :] += lax.dot_general(xsb, xsb, (((0,), (0,)), ((), ())),
                                 preferred_element_type=jnp.float32)


def _run_gram(xs):
    return pl.pallas_call(
        _gram_body,
        grid=(NGB,),
        in_specs=[pl.BlockSpec((GB, 2 * D), lambda b: (b, 0))],
        out_specs=pl.BlockSpec((D, D), lambda b: (b, 0)),
        out_shape=jax.ShapeDtypeStruct((NGB * D, D), jnp.float32),
        scratch_shapes=[pltpu.VMEM((D, D), jnp.float32)],
        compiler_params=pltpu.CompilerParams(
            dimension_semantics=("arbitrary",),
        ),
    )(xs)


def _stage_d(bep_ref, mlim_ref, xs_ref, p_ref,
             counts_ref, sums_ref, ft_ref, mt_ref, ct_ref, out_ref,
             sedge, prevt):
    k = pl.program_id(0)

    @pl.when(k == 0)
    def _init():
        prevt[:, :] = jnp.zeros_like(prevt)

    xsb = xs_ref[:, :D]
    mlim = mlim_ref[k]
    msk = (lax.broadcasted_iota(jnp.int32, (GB, 1), 0) < mlim
           ).astype(jnp.float32)
    xs_bf = xsb.astype(jnp.bfloat16)
    xm_bf = (xsb * msk).astype(jnp.bfloat16)
    pe = lax.dot_general(xm_bf, xs_bf, (((0,), (0,)), ((), ())),
                         preferred_element_type=jnp.float32)
    tk = p_ref[:, :] + pe
    sedge[pl.ds(k * D, D), :] = tk - prevt[:, :]
    prevt[:, :] = tk

    @pl.when(k == K - 1)
    def _epilogue():
        s_flat = sedge[:, :]
        counts = counts_ref[:, :]
        safe = jnp.maximum(counts, 1.0)
        means = sums_ref[:, :] / safe

        filling = counts / jnp.float32(N)
        loss_fil = jnp.sum((filling - ft_ref[:, :]) ** 2,
                           axis=(0, 1), keepdims=True) / jnp.float32(K)
        loss_means = jnp.sum((means - mt_ref[:, :]) ** 2,
                             axis=(0, 1), keepdims=True) / jnp.float32(K * D)

        m3 = jnp.reshape(jnp.broadcast_to(means[:, None, :], (K, D, D)),
                         (K * D, D))
        rio = lax.broadcasted_iota(jnp.int32, (K * D, D), 0)
        jio = lax.broadcasted_iota(jnp.int32, (K * D, D), 1)
        isel = (rio % D == jio).astype(jnp.float32)
        m4 = jnp.sum(m3 * isel, axis=1, keepdims=True)

        countsb = jnp.reshape(jnp.broadcast_to(counts[:, :, None], (K, D, 1)),
                              (K * D, 1))
        denomb = jnp.maximum(countsb - 1.0, 1.0)
        covs = (s_flat - countsb * (m4 * m3)) / denomb
        loss_covs = jnp.sum((covs - ct_ref[:, :]) ** 2,
                            axis=(0, 1), keepdims=True) / jnp.float32(K * D * D)

        out_ref[:, :] = loss_fil + loss_means + loss_covs


def _run_stage_d(bep, mlim, xs, pgram, counts, sums, ft, mt, ct):
    grid_spec = pltpu.PrefetchScalarGridSpec(
        num_scalar_prefetch=2,
        grid=(K,),
        in_specs=[
            pl.BlockSpec((GB, 2 * D), lambda k, bep, mlim: (bep[k], 0)),
            pl.BlockSpec((D, D), lambda k, bep, mlim: (bep[k], 0)),
            pl.BlockSpec((K, 1), lambda k, bep, mlim: (0, 0)),
            pl.BlockSpec((K, D), lambda k, bep, mlim: (0, 0)),
            pl.BlockSpec((K, 1), lambda k, bep, mlim: (0, 0)),
            pl.BlockSpec((K, D), lambda k, bep, mlim: (0, 0)),
            pl.BlockSpec((K * D, D), lambda k, bep, mlim: (0, 0)),
        ],
        out_specs=pl.BlockSpec((1, 1), lambda k, bep, mlim: (0, 0)),
        scratch_shapes=[
            pltpu.VMEM((K * D, D), jnp.float32),
            pltpu.VMEM((D, D), jnp.float32),
        ],
    )
    return pl.pallas_call(
        _stage_d,
        grid_spec=grid_spec,
        out_shape=jax.ShapeDtypeStruct((1, 1), jnp.float32),
        compiler_params=pltpu.CompilerParams(
            dimension_semantics=("arbitrary",),
        ),
    )(bep, mlim, xs, pgram, counts, sums, ft, mt, ct)


def kernel(x, cluster_centers, filling_target, means_target, covs_target):
    jio = lax.broadcasted_iota(jnp.int32, (BLK, BLK), 1)
    iio = lax.broadcasted_iota(jnp.int32, (BLK, BLK), 0)
    lt = (iio < jio).astype(jnp.bfloat16)

    pos3, counts, sums, off, xp = _run_stage_a(x, cluster_centers, lt)

    pos_i = pos3.reshape(N)
    off_i = off.astype(jnp.int32).reshape(K)
    xs = _scatter_rows(xp, pos_i)

    pgram = _run_gram(xs)

    e_i = off_i + counts.astype(jnp.int32).reshape(K)
    bep = jnp.clip(e_i // GB, 0, NGB - 1).astype(jnp.int32)
    mlim = (e_i - bep * GB).astype(jnp.int32)

    ft = filling_target.reshape(K, 1)
    ct = covs_target.reshape(K * D, D)
    out = _run_stage_d(bep, mlim, xs, pgram, counts, sums,
                       ft, means_target, ct)
    return out[0, 0]


# speedup vs baseline: 2.1409x; 2.1162x over previous
import functools

import jax
import jax.numpy as jnp
from jax.experimental import pallas as pl
from jax.experimental.pallas import tpu as pltpu

N, K, D = 16384, 64, 64
BLK = 512


def _loss_kernel(xt_ref, x_ref, c_ref, ft_ref, mt_ref, ct_ref, out_ref,
                 counts_acc, sums_acc, s_acc, *, nblk):
    i = pl.program_id(0)

    @pl.when(i == 0)
    def _init():
        counts_acc[:, :] = jnp.zeros_like(counts_acc)
        sums_acc[:, :] = jnp.zeros_like(sums_acc)
        s_acc[:, :] = jnp.zeros_like(s_acc)

    xt = xt_ref[0]
    xb = x_ref[:, :]
    c = c_ref[:, :]

    cn = jnp.sum(c * c, axis=1, keepdims=True)
    xn = jnp.sum(xt * xt, axis=0, keepdims=True)
    d2 = cn - 2.0 * jnp.dot(c, xt, preferred_element_type=jnp.float32) + xn

    dmin = jnp.min(d2, axis=0, keepdims=True)
    kio = jax.lax.broadcasted_iota(jnp.int32, (K, BLK), 0)
    pred = jnp.min(jnp.where(d2 <= dmin, kio, K), axis=0, keepdims=True)
    onehot = (kio == pred).astype(jnp.float32)

    counts_acc[:, :] += jnp.sum(onehot, axis=1, keepdims=True)

    oh_bf = onehot.astype(jnp.bfloat16)
    xb_bf = xb.astype(jnp.bfloat16)
    xt_bf = xt.astype(jnp.bfloat16)

    sums_acc[:, :] += jnp.dot(oh_bf, xb_bf,
                              preferred_element_type=jnp.float32)

    m_oh = jnp.reshape(jnp.broadcast_to(oh_bf[:, None, :], (K, D, BLK)),
                       (K * D, BLK))
    m_xt = jnp.reshape(jnp.broadcast_to(xt_bf[None, :, :], (K, D, BLK)),
                       (K * D, BLK))
    s_acc[:, :] += jnp.dot(m_oh * m_xt, xb_bf,
                           preferred_element_type=jnp.float32)

    @pl.when(i == nblk - 1)
    def _epilogue():
        counts = counts_acc[:, :]
        safe = jnp.maximum(counts, 1.0)
        means = sums_acc[:, :] / safe

        filling = counts / jnp.float32(N)
        loss_fil = jnp.sum((filling - ft_ref[:, :]) ** 2,
                           axis=(0, 1), keepdims=True) / jnp.float32(K)
        loss_means = jnp.sum((means - mt_ref[:, :]) ** 2,
                             axis=(0, 1), keepdims=True) / jnp.float32(K * D)

        m3 = jnp.reshape(jnp.broadcast_to(means[:, None, :], (K, D, D)),
                         (K * D, D))
        rio = jax.lax.broadcasted_iota(jnp.int32, (K * D, D), 0)
        jio = jax.lax.broadcasted_iota(jnp.int32, (K * D, D), 1)
        isel = (rio % D == jio).astype(jnp.float32)
        m4 = jnp.sum(m3 * isel, axis=1, keepdims=True)

        countsb = jnp.reshape(jnp.broadcast_to(counts[:, :, None], (K, D, 1)),
                              (K * D, 1))
        denomb = jnp.maximum(countsb - 1.0, 1.0)
        covs = (s_acc[:, :] - countsb * (m4 * m3)) / denomb
        loss_covs = jnp.sum((covs - ct_ref[:, :]) ** 2,
                            axis=(0, 1), keepdims=True) / jnp.float32(K * D * D)

        out_ref[:, :] = loss_fil + loss_means + loss_covs


def kernel(x, cluster_centers, filling_target, means_target, covs_target):
    nblk = N // BLK
    xt = jnp.transpose(x.reshape(nblk, BLK, D), (0, 2, 1))
    ft = filling_target.reshape(K, 1)
    ct = covs_target.reshape(K * D, D)

    out = pl.pallas_call(
        functools.partial(_loss_kernel, nblk=nblk),
        grid=(nblk,),
        in_specs=[
            pl.BlockSpec((1, D, BLK), lambda i: (i, 0, 0)),
            pl.BlockSpec((BLK, D), lambda i: (i, 0)),
            pl.BlockSpec((K, D), lambda i: (0, 0)),
            pl.BlockSpec((K, 1), lambda i: (0, 0)),
            pl.BlockSpec((K, D), lambda i: (0, 0)),
            pl.BlockSpec((K * D, D), lambda i: (0, 0)),
        ],
        out_specs=pl.BlockSpec((1, 1), lambda i: (0, 0)),
        out_shape=jax.ShapeDtypeStruct((1, 1), jnp.float32),
        scratch_shapes=[
            pltpu.VMEM((K, 1), jnp.float32),
            pltpu.VMEM((K, D), jnp.float32),
            pltpu.VMEM((K * D, D), jnp.float32),
        ],
        compiler_params=pltpu.CompilerParams(
            dimension_semantics=("arbitrary",),
        ),
    )(xt, x, cluster_centers, ft, means_target, ct)
    return out[0, 0]


# speedup vs baseline: 3.1517x; 1.4722x over previous
import functools

import jax
import jax.numpy as jnp
from jax.experimental import pallas as pl
from jax.experimental.pallas import tpu as pltpu

N, K, D = 16384, 64, 64
BLK = 512


def _loss_kernel(xt_ref, x_ref, c_ref, ft_ref, mt_ref, ct_ref, out_ref,
                 counts_acc, sums_acc, s_acc, *, nblk):
    i = pl.program_id(0)

    @pl.when(i == 0)
    def _init():
        counts_acc[:, :] = jnp.zeros_like(counts_acc)
        sums_acc[:, :] = jnp.zeros_like(sums_acc)
        s_acc[:, :] = jnp.zeros_like(s_acc)

    xt = xt_ref[:, :]
    xb = x_ref[:, :]
    c = c_ref[:, :]

    cn = jnp.sum(c * c, axis=1, keepdims=True)
    xn = jnp.sum(xt * xt, axis=0, keepdims=True)
    d2 = cn - 2.0 * jnp.dot(c, xt, preferred_element_type=jnp.float32) + xn

    dmin = jnp.min(d2, axis=0, keepdims=True)
    kio = jax.lax.broadcasted_iota(jnp.int32, (K, BLK), 0)
    pred = jnp.min(jnp.where(d2 <= dmin, kio, K), axis=0, keepdims=True)
    onehot = (kio == pred).astype(jnp.float32)

    counts_acc[:, :] += jnp.sum(onehot, axis=1, keepdims=True)

    oh_bf = onehot.astype(jnp.bfloat16)
    xb_bf = xb.astype(jnp.bfloat16)

    sums_acc[:, :] += jnp.dot(oh_bf, xb_bf,
                              preferred_element_type=jnp.float32)

    ohm = kio == pred
    xt_f8 = xt.astype(jnp.float8_e4m3fn)
    xb_f8 = xb.astype(jnp.float8_e4m3fn)
    m_oh = jnp.reshape(jnp.broadcast_to(ohm[:, None, :], (K, D, BLK)),
                       (K * D, BLK))
    m_xt = jnp.reshape(jnp.broadcast_to(xt_f8[None, :, :], (K, D, BLK)),
                       (K * D, BLK))
    m2 = jnp.where(m_oh, m_xt, jnp.float8_e4m3fn(0.0))
    s_acc[:, :] += jnp.dot(m2, xb_f8,
                           preferred_element_type=jnp.float32)

    @pl.when(i == nblk - 1)
    def _epilogue():
        counts = counts_acc[:, :]
        safe = jnp.maximum(counts, 1.0)
        means = sums_acc[:, :] / safe

        filling = counts / jnp.float32(N)
        loss_fil = jnp.sum((filling - ft_ref[:, :]) ** 2,
                           axis=(0, 1), keepdims=True) / jnp.float32(K)
        loss_means = jnp.sum((means - mt_ref[:, :]) ** 2,
                             axis=(0, 1), keepdims=True) / jnp.float32(K * D)

        m3 = jnp.reshape(jnp.broadcast_to(means[:, None, :], (K, D, D)),
                         (K * D, D))
        rio = jax.lax.broadcasted_iota(jnp.int32, (K * D, D), 0)
        jio = jax.lax.broadcasted_iota(jnp.int32, (K * D, D), 1)
        isel = (rio % D == jio).astype(jnp.float32)
        m4 = jnp.sum(m3 * isel, axis=1, keepdims=True)

        countsb = jnp.reshape(jnp.broadcast_to(counts[:, :, None], (K, D, 1)),
                              (K * D, 1))
        denomb = jnp.maximum(countsb - 1.0, 1.0)
        covs = (s_acc[:, :] - countsb * (m4 * m3)) / denomb
        loss_covs = jnp.sum((covs - ct_ref[:, :]) ** 2,
                            axis=(0, 1), keepdims=True) / jnp.float32(K * D * D)

        out_ref[:, :] = loss_fil + loss_means + loss_covs


def kernel(x, cluster_centers, filling_target, means_target, covs_target):
    nblk = N // BLK
    xt = x.T
    ft = filling_target.reshape(K, 1)
    ct = covs_target.reshape(K * D, D)

    out = pl.pallas_call(
        functools.partial(_loss_kernel, nblk=nblk),
        grid=(nblk,),
        in_specs=[
            pl.BlockSpec((D, BLK), lambda i: (0, i)),
            pl.BlockSpec((BLK, D), lambda i: (i, 0)),
            pl.BlockSpec((K, D), lambda i: (0, 0)),
            pl.BlockSpec((K, 1), lambda i: (0, 0)),
            pl.BlockSpec((K, D), lambda i: (0, 0)),
            pl.BlockSpec((K * D, D), lambda i: (0, 0)),
        ],
        out_specs=pl.BlockSpec((1, 1), lambda i: (0, 0)),
        out_shape=jax.ShapeDtypeStruct((1, 1), jnp.float32),
        scratch_shapes=[
            pltpu.VMEM((K, 1), jnp.float32),
            pltpu.VMEM((K, D), jnp.float32),
            pltpu.VMEM((K * D, D), jnp.float32),
        ],
        compiler_params=pltpu.CompilerParams(
            dimension_semantics=("arbitrary",),
        ),
    )(xt, x, cluster_centers, ft, means_target, ct)
    return out[0, 0]


# speedup vs baseline: 3.6003x; 1.1423x over previous
import functools

import jax
import jax.numpy as jnp
from jax.experimental import pallas as pl
from jax.experimental.pallas import tpu as pltpu

N, K, D = 16384, 64, 64
BLK = 1024


def _loss_kernel(xt_ref, x_ref, c_ref, ft_ref, mt_ref, ct_ref, out_ref,
                 counts_acc, sums_acc, s_acc, *, nblk):
    i = pl.program_id(0)

    @pl.when(i == 0)
    def _init():
        counts_acc[:, :] = jnp.zeros_like(counts_acc)
        sums_acc[:, :] = jnp.zeros_like(sums_acc)
        s_acc[:, :] = jnp.zeros_like(s_acc)

    xt = xt_ref[:, :]
    xb = x_ref[:, :]
    c = c_ref[:, :]

    cn = jnp.sum(c * c, axis=1, keepdims=True)
    xn = jnp.sum(xt * xt, axis=0, keepdims=True)
    d2 = cn - 2.0 * jnp.dot(c, xt, preferred_element_type=jnp.float32) + xn

    dmin = jnp.min(d2, axis=0, keepdims=True)
    kio = jax.lax.broadcasted_iota(jnp.int32, (K, BLK), 0)
    pred = jnp.min(jnp.where(d2 <= dmin, kio, K), axis=0, keepdims=True)
    onehot = (kio == pred).astype(jnp.float32)

    counts_acc[:, :] += jnp.sum(onehot, axis=1, keepdims=True)

    oh_bf = onehot.astype(jnp.bfloat16)
    xb_bf = xb.astype(jnp.bfloat16)

    sums_acc[:, :] += jnp.dot(oh_bf, xb_bf,
                              preferred_element_type=jnp.float32)

    ohm = kio == pred
    xt_f8 = xt.astype(jnp.float8_e4m3fn)
    xb_f8 = xb.astype(jnp.float8_e4m3fn)
    m_oh = jnp.reshape(jnp.broadcast_to(ohm[:, None, :], (K, D, BLK)),
                       (K * D, BLK))
    m_xt = jnp.reshape(jnp.broadcast_to(xt_f8[None, :, :], (K, D, BLK)),
                       (K * D, BLK))
    m2 = jnp.where(m_oh, m_xt, jnp.float8_e4m3fn(0.0))
    s_acc[:, :] += jnp.dot(m2, xb_f8,
                           preferred_element_type=jnp.float32)

    @pl.when(i == nblk - 1)
    def _epilogue():
        counts = counts_acc[:, :]
        safe = jnp.maximum(counts, 1.0)
        means = sums_acc[:, :] / safe

        filling = counts / jnp.float32(N)
        loss_fil = jnp.sum((filling - ft_ref[:, :]) ** 2,
                           axis=(0, 1), keepdims=True) / jnp.float32(K)
        loss_means = jnp.sum((means - mt_ref[:, :]) ** 2,
                             axis=(0, 1), keepdims=True) / jnp.float32(K * D)

        m3 = jnp.reshape(jnp.broadcast_to(means[:, None, :], (K, D, D)),
                         (K * D, D))
        rio = jax.lax.broadcasted_iota(jnp.int32, (K * D, D), 0)
        jio = jax.lax.broadcasted_iota(jnp.int32, (K * D, D), 1)
        isel = (rio % D == jio).astype(jnp.float32)
        m4 = jnp.sum(m3 * isel, axis=1, keepdims=True)

        countsb = jnp.reshape(jnp.broadcast_to(counts[:, :, None], (K, D, 1)),
                              (K * D, 1))
        denomb = jnp.maximum(countsb - 1.0, 1.0)
        covs = (s_acc[:, :] - countsb * (m4 * m3)) / denomb
        loss_covs = jnp.sum((covs - ct_ref[:, :]) ** 2,
                            axis=(0, 1), keepdims=True) / jnp.float32(K * D * D)

        out_ref[:, :] = loss_fil + loss_means + loss_covs


def kernel(x, cluster_centers, filling_target, means_target, covs_target):
    nblk = N // BLK
    xt = x.T
    ft = filling_target.reshape(K, 1)
    ct = covs_target.reshape(K * D, D)

    out = pl.pallas_call(
        functools.partial(_loss_kernel, nblk=nblk),
        grid=(nblk,),
        in_specs=[
            pl.BlockSpec((D, BLK), lambda i: (0, i)),
            pl.BlockSpec((BLK, D), lambda i: (i, 0)),
            pl.BlockSpec((K, D), lambda i: (0, 0)),
            pl.BlockSpec((K, 1), lambda i: (0, 0)),
            pl.BlockSpec((K, D), lambda i: (0, 0)),
            pl.BlockSpec((K * D, D), lambda i: (0, 0)),
        ],
        out_specs=pl.BlockSpec((1, 1), lambda i: (0, 0)),
        out_shape=jax.ShapeDtypeStruct((1, 1), jnp.float32),
        scratch_shapes=[
            pltpu.VMEM((K, 1), jnp.float32),
            pltpu.VMEM((K, D), jnp.float32),
            pltpu.VMEM((K * D, D), jnp.float32),
        ],
        compiler_params=pltpu.CompilerParams(
            dimension_semantics=("arbitrary",),
        ),
    )(xt, x, cluster_centers, ft, means_target, ct)
    return out[0, 0]


# speedup vs baseline: 3.8276x; 1.0631x over previous
import functools

import jax
import jax.numpy as jnp
from jax.experimental import pallas as pl
from jax.experimental.pallas import tpu as pltpu

N, K, D = 16384, 64, 64
BLK = 2048


def _loss_kernel(xt_ref, x_ref, c_ref, ft_ref, mt_ref, ct_ref, out_ref,
                 counts_acc, sums_acc, s_acc, *, nblk):
    i = pl.program_id(0)

    @pl.when(i == 0)
    def _init():
        counts_acc[:, :] = jnp.zeros_like(counts_acc)
        sums_acc[:, :] = jnp.zeros_like(sums_acc)
        s_acc[:, :] = jnp.zeros_like(s_acc)

    xt = xt_ref[:, :]
    xb = x_ref[:, :]
    c = c_ref[:, :]

    cn = jnp.sum(c * c, axis=1, keepdims=True)
    xn = jnp.sum(xt * xt, axis=0, keepdims=True)
    d2 = cn - 2.0 * jnp.dot(c, xt, preferred_element_type=jnp.float32) + xn

    dmin = jnp.min(d2, axis=0, keepdims=True)
    kio = jax.lax.broadcasted_iota(jnp.int32, (K, BLK), 0)
    pred = jnp.min(jnp.where(d2 <= dmin, kio, K), axis=0, keepdims=True)
    onehot = (kio == pred).astype(jnp.float32)

    counts_acc[:, :] += jnp.sum(onehot, axis=1, keepdims=True)

    oh_bf = onehot.astype(jnp.bfloat16)
    xb_bf = xb.astype(jnp.bfloat16)

    sums_acc[:, :] += jnp.dot(oh_bf, xb_bf,
                              preferred_element_type=jnp.float32)

    ohm = kio == pred
    xt_f8 = xt.astype(jnp.float8_e4m3fn)
    xb_f8 = xb.astype(jnp.float8_e4m3fn)
    m_oh = jnp.reshape(jnp.broadcast_to(ohm[:, None, :], (K, D, BLK)),
                       (K * D, BLK))
    m_xt = jnp.reshape(jnp.broadcast_to(xt_f8[None, :, :], (K, D, BLK)),
                       (K * D, BLK))
    m2 = jnp.where(m_oh, m_xt, jnp.float8_e4m3fn(0.0))
    s_acc[:, :] += jnp.dot(m2, xb_f8,
                           preferred_element_type=jnp.float32)

    @pl.when(i == nblk - 1)
    def _epilogue():
        counts = counts_acc[:, :]
        safe = jnp.maximum(counts, 1.0)
        means = sums_acc[:, :] / safe

        filling = counts / jnp.float32(N)
        loss_fil = jnp.sum((filling - ft_ref[:, :]) ** 2,
                           axis=(0, 1), keepdims=True) / jnp.float32(K)
        loss_means = jnp.sum((means - mt_ref[:, :]) ** 2,
                             axis=(0, 1), keepdims=True) / jnp.float32(K * D)

        m3 = jnp.reshape(jnp.broadcast_to(means[:, None, :], (K, D, D)),
                         (K * D, D))
        rio = jax.lax.broadcasted_iota(jnp.int32, (K * D, D), 0)
        jio = jax.lax.broadcasted_iota(jnp.int32, (K * D, D), 1)
        isel = (rio % D == jio).astype(jnp.float32)
        m4 = jnp.sum(m3 * isel, axis=1, keepdims=True)

        countsb = jnp.reshape(jnp.broadcast_to(counts[:, :, None], (K, D, 1)),
                              (K * D, 1))
        denomb = jnp.maximum(countsb - 1.0, 1.0)
        covs = (s_acc[:, :] - countsb * (m4 * m3)) / denomb
        loss_covs = jnp.sum((covs - ct_ref[:, :]) ** 2,
                            axis=(0, 1), keepdims=True) / jnp.float32(K * D * D)

        out_ref[:, :] = loss_fil + loss_means + loss_covs


def kernel(x, cluster_centers, filling_target, means_target, covs_target):
    nblk = N // BLK
    xt = x.T
    ft = filling_target.reshape(K, 1)
    ct = covs_target.reshape(K * D, D)

    out = pl.pallas_call(
        functools.partial(_loss_kernel, nblk=nblk),
        grid=(nblk,),
        in_specs=[
            pl.BlockSpec((D, BLK), lambda i: (0, i)),
            pl.BlockSpec((BLK, D), lambda i: (i, 0)),
            pl.BlockSpec((K, D), lambda i: (0, 0)),
            pl.BlockSpec((K, 1), lambda i: (0, 0)),
            pl.BlockSpec((K, D), lambda i: (0, 0)),
            pl.BlockSpec((K * D, D), lambda i: (0, 0)),
        ],
        out_specs=pl.BlockSpec((1, 1), lambda i: (0, 0)),
        out_shape=jax.ShapeDtypeStruct((1, 1), jnp.float32),
        scratch_shapes=[
            pltpu.VMEM((K, 1), jnp.float32),
            pltpu.VMEM((K, D), jnp.float32),
            pltpu.VMEM((K * D, D), jnp.float32),
        ],
        compiler_params=pltpu.CompilerParams(
            dimension_semantics=("arbitrary",),
        ),
    )(xt, x, cluster_centers, ft, means_target, ct)
    return out[0, 0]


# speedup vs baseline: 3.8882x; 1.0158x over previous
import functools

import jax
import jax.numpy as jnp
from jax.experimental import pallas as pl
from jax.experimental.pallas import tpu as pltpu

N, K, D = 16384, 64, 64
BLK = 4096


def _loss_kernel(xt_ref, x_ref, c_ref, ft_ref, mt_ref, ct_ref, out_ref,
                 counts_acc, sums_acc, s_acc, *, nblk):
    i = pl.program_id(0)

    @pl.when(i == 0)
    def _init():
        counts_acc[:, :] = jnp.zeros_like(counts_acc)
        sums_acc[:, :] = jnp.zeros_like(sums_acc)
        s_acc[:, :] = jnp.zeros_like(s_acc)

    xt = xt_ref[:, :]
    xb = x_ref[:, :]
    c = c_ref[:, :]

    cn = jnp.sum(c * c, axis=1, keepdims=True)
    xn = jnp.sum(xt * xt, axis=0, keepdims=True)
    d2 = cn - 2.0 * jnp.dot(c, xt, preferred_element_type=jnp.float32) + xn

    dmin = jnp.min(d2, axis=0, keepdims=True)
    kio = jax.lax.broadcasted_iota(jnp.int32, (K, BLK), 0)
    pred = jnp.min(jnp.where(d2 <= dmin, kio, K), axis=0, keepdims=True)
    onehot = (kio == pred).astype(jnp.float32)

    counts_acc[:, :] += jnp.sum(onehot, axis=1, keepdims=True)

    oh_bf = onehot.astype(jnp.bfloat16)
    xb_bf = xb.astype(jnp.bfloat16)

    sums_acc[:, :] += jnp.dot(oh_bf, xb_bf,
                              preferred_element_type=jnp.float32)

    ohm = kio == pred
    xt_f8 = xt.astype(jnp.float8_e4m3fn)
    xb_f8 = xb.astype(jnp.float8_e4m3fn)
    m_oh = jnp.reshape(jnp.broadcast_to(ohm[:, None, :], (K, D, BLK)),
                       (K * D, BLK))
    m_xt = jnp.reshape(jnp.broadcast_to(xt_f8[None, :, :], (K, D, BLK)),
                       (K * D, BLK))
    m2 = jnp.where(m_oh, m_xt, jnp.float8_e4m3fn(0.0))
    s_acc[:, :] += jnp.dot(m2, xb_f8,
                           preferred_element_type=jnp.float32)

    @pl.when(i == nblk - 1)
    def _epilogue():
        counts = counts_acc[:, :]
        safe = jnp.maximum(counts, 1.0)
        means = sums_acc[:, :] / safe

        filling = counts / jnp.float32(N)
        loss_fil = jnp.sum((filling - ft_ref[:, :]) ** 2,
                           axis=(0, 1), keepdims=True) / jnp.float32(K)
        loss_means = jnp.sum((means - mt_ref[:, :]) ** 2,
                             axis=(0, 1), keepdims=True) / jnp.float32(K * D)

        m3 = jnp.reshape(jnp.broadcast_to(means[:, None, :], (K, D, D)),
                         (K * D, D))
        rio = jax.lax.broadcasted_iota(jnp.int32, (K * D, D), 0)
        jio = jax.lax.broadcasted_iota(jnp.int32, (K * D, D), 1)
        isel = (rio % D == jio).astype(jnp.float32)
        m4 = jnp.sum(m3 * isel, axis=1, keepdims=True)

        countsb = jnp.reshape(jnp.broadcast_to(counts[:, :, None], (K, D, 1)),
                              (K * D, 1))
        denomb = jnp.maximum(countsb - 1.0, 1.0)
        covs = (s_acc[:, :] - countsb * (m4 * m3)) / denomb
        loss_covs = jnp.sum((covs - ct_ref[:, :]) ** 2,
                            axis=(0, 1), keepdims=True) / jnp.float32(K * D * D)

        out_ref[:, :] = loss_fil + loss_means + loss_covs


def kernel(x, cluster_centers, filling_target, means_target, covs_target):
    nblk = N // BLK
    xt = x.T
    ft = filling_target.reshape(K, 1)
    ct = covs_target.reshape(K * D, D)

    out = pl.pallas_call(
        functools.partial(_loss_kernel, nblk=nblk),
        grid=(nblk,),
        in_specs=[
            pl.BlockSpec((D, BLK), lambda i: (0, i)),
            pl.BlockSpec((BLK, D), lambda i: (i, 0)),
            pl.BlockSpec((K, D), lambda i: (0, 0)),
            pl.BlockSpec((K, 1), lambda i: (0, 0)),
            pl.BlockSpec((K, D), lambda i: (0, 0)),
            pl.BlockSpec((K * D, D), lambda i: (0, 0)),
        ],
        out_specs=pl.BlockSpec((1, 1), lambda i: (0, 0)),
        out_shape=jax.ShapeDtypeStruct((1, 1), jnp.float32),
        scratch_shapes=[
            pltpu.VMEM((K, 1), jnp.float32),
            pltpu.VMEM((K, D), jnp.float32),
            pltpu.VMEM((K * D, D), jnp.float32),
        ],
        compiler_params=pltpu.CompilerParams(
            dimension_semantics=("arbitrary",),
        ),
    )(xt, x, cluster_centers, ft, means_target, ct)
    return out[0, 0]
